# Initial kernel scaffold; baseline (speedup 1.0000x reference)
import jax
import jax.numpy as jnp
from jax.experimental import pallas as pl


def kernel(sequence_tensor, span_indices, width_embedding):
    raise NotImplementedError("write your pallas kernel here")


# speedup vs baseline: 3672.7899x; 3672.7899x over previous
import functools

import jax
import jax.numpy as jnp
from jax import lax
from jax.experimental import pallas as pl
from jax.experimental.pallas import tpu as pltpu
from jax.experimental.pallas import tpu_sc as plsc

B, S, D = 16, 2048, 256
N = 4096
WD = 32
OUTW = 2 * D + WD

NW = 32
SPW = (B * N) // NW
C = 128
NCHUNK = SPW // C
LANES = 16


def _body(seq_hbm, starts_hbm, ends_hbm, wtab_hbm, out_hbm,
          sidx_v, eidx_v, widx_v, srow_v, erow_v, wrow_v,
          sem_s, sem_e, sem_w):
    cid = lax.axis_index("c")
    sid = lax.axis_index("s")
    wid = sid * 2 + cid
    rowoff = (wid // 2) * S

    def chunk_body(k, carry):
        base = wid * SPW + k * C
        pltpu.sync_copy(starts_hbm.at[pl.ds(base, C)], sidx_v)
        pltpu.sync_copy(ends_hbm.at[pl.ds(base, C)], eidx_v)
        for j in range(C // LANES):
            sl = pl.ds(j * LANES, LANES)
            starts = sidx_v[sl]
            ends = eidx_v[sl]
            widx_v[sl] = ends - starts
            sidx_v[sl] = starts + rowoff
            eidx_v[sl] = ends + rowoff
        cps = pltpu.async_copy(seq_hbm.at[sidx_v], srow_v, sem_s)
        cpe = pltpu.async_copy(seq_hbm.at[eidx_v], erow_v, sem_e)
        cpw = pltpu.async_copy(wtab_hbm.at[widx_v], wrow_v, sem_w)
        cps.wait()
        cpe.wait()
        cpw.wait()
        pltpu.sync_copy(srow_v, out_hbm.at[pl.ds(base, C), pl.ds(0, D)])
        pltpu.sync_copy(erow_v, out_hbm.at[pl.ds(base, C), pl.ds(D, D)])
        pltpu.sync_copy(wrow_v, out_hbm.at[pl.ds(base, C), pl.ds(2 * D, WD)])
        return carry

    lax.fori_loop(0, NCHUNK, chunk_body, 0)


_span_kernel = functools.partial(
    pl.kernel,
    mesh=plsc.VectorSubcoreMesh(core_axis_name="c", subcore_axis_name="s"),
    compiler_params=pltpu.CompilerParams(use_tc_tiling_on_sc=False),
    out_type=jax.ShapeDtypeStruct((B * N, OUTW), jnp.float32),
    scratch_types=[
        pltpu.VMEM((C,), jnp.int32),
        pltpu.VMEM((C,), jnp.int32),
        pltpu.VMEM((C,), jnp.int32),
        pltpu.VMEM((C, D), jnp.float32),
        pltpu.VMEM((C, D), jnp.float32),
        pltpu.VMEM((C, WD), jnp.float32),
        pltpu.SemaphoreType.DMA,
        pltpu.SemaphoreType.DMA,
        pltpu.SemaphoreType.DMA,
    ],
)(_body)


def kernel(sequence_tensor, span_indices, width_embedding):
    seq_flat = sequence_tensor.reshape(B * S, D)
    sp32 = span_indices.astype(jnp.int32)
    starts_flat = sp32[..., 0].reshape(B * N)
    ends_flat = sp32[..., 1].reshape(B * N)
    out = _span_kernel(seq_flat, starts_flat, ends_flat, width_embedding)
    return out.reshape(B, N, OUTW)


# speedup vs baseline: 3816.0210x; 1.0390x over previous
import functools

import jax
import jax.numpy as jnp
from jax import lax
from jax.experimental import pallas as pl
from jax.experimental.pallas import tpu as pltpu
from jax.experimental.pallas import tpu_sc as plsc

B, S, D = 16, 2048, 256
N = 4096
WD = 32
OUTW = 2 * D + WD

NW = 32
SPW = (B * N) // NW
C = 64
NCHUNK = SPW // C
LANES = 16


def _body(seq_hbm, starts_hbm, ends_hbm, wtab_hbm, out_hbm,
          sidx0, sidx1, eidx0, eidx1, widx0, widx1,
          srow0, srow1, erow0, erow1, wrow0, wrow1,
          sem_g0, sem_g1, sem_w0, sem_w1):
    cid = lax.axis_index("c")
    sid = lax.axis_index("s")
    wid = sid * 2 + cid
    rowoff = (wid // 2) * S
    first = wid * SPW

    sidx = (sidx0, sidx1)
    eidx = (eidx0, eidx1)
    widx = (widx0, widx1)
    srow = (srow0, srow1)
    erow = (erow0, erow1)
    wrow = (wrow0, wrow1)
    sem_g = (sem_g0, sem_g1)
    sem_w = (sem_w0, sem_w1)

    def fire_gathers(k, b):
        base = first + k * C
        pltpu.sync_copy(starts_hbm.at[pl.ds(base, C)], sidx[b])
        pltpu.sync_copy(ends_hbm.at[pl.ds(base, C)], eidx[b])
        for j in range(C // LANES):
            sl = pl.ds(j * LANES, LANES)
            s = sidx[b][sl]
            e = eidx[b][sl]
            widx[b][sl] = e - s
            sidx[b][sl] = s + rowoff
            eidx[b][sl] = e + rowoff
        pltpu.async_copy(seq_hbm.at[sidx[b]], srow[b], sem_g[b])
        pltpu.async_copy(seq_hbm.at[eidx[b]], erow[b], sem_g[b])
        pltpu.async_copy(wtab_hbm.at[widx[b]], wrow[b], sem_g[b])

    def wait_gathers(b):
        pltpu.make_async_copy(seq_hbm.at[sidx[b]], srow[b], sem_g[b]).wait()
        pltpu.make_async_copy(seq_hbm.at[eidx[b]], erow[b], sem_g[b]).wait()
        pltpu.make_async_copy(wtab_hbm.at[widx[b]], wrow[b], sem_g[b]).wait()

    def fire_writes(k, b):
        base = first + k * C
        pltpu.async_copy(srow[b], out_hbm.at[pl.ds(base, C), pl.ds(0, D)],
                         sem_w[b])
        pltpu.async_copy(erow[b], out_hbm.at[pl.ds(base, C), pl.ds(D, D)],
                         sem_w[b])
        pltpu.async_copy(wrow[b], out_hbm.at[pl.ds(base, C), pl.ds(2 * D, WD)],
                         sem_w[b])

    def wait_writes(k, b):
        base = first + k * C
        pltpu.make_async_copy(
            srow[b], out_hbm.at[pl.ds(base, C), pl.ds(0, D)], sem_w[b]).wait()
        pltpu.make_async_copy(
            erow[b], out_hbm.at[pl.ds(base, C), pl.ds(D, D)], sem_w[b]).wait()
        pltpu.make_async_copy(
            wrow[b], out_hbm.at[pl.ds(base, C), pl.ds(2 * D, WD)],
            sem_w[b]).wait()

    fire_gathers(0, 0)
    fire_gathers(1, 1)
    wait_gathers(0)
    fire_writes(0, 0)

    def pair_body(p, carry):
        k1 = 2 * p + 1
        wait_writes(k1 - 1, 0)
        fire_gathers(k1 + 1, 0)
        wait_gathers(1)
        fire_writes(k1, 1)

        k2 = 2 * p + 2
        wait_writes(k2 - 1, 1)
        fire_gathers(k2 + 1, 1)
        wait_gathers(0)
        fire_writes(k2, 0)
        return carry

    lax.fori_loop(0, (NCHUNK - 2) // 2, pair_body, 0)

    wait_gathers(1)
    fire_writes(NCHUNK - 1, 1)
    wait_writes(NCHUNK - 2, 0)
    wait_writes(NCHUNK - 1, 1)


_span_kernel = functools.partial(
    pl.kernel,
    mesh=plsc.VectorSubcoreMesh(core_axis_name="c", subcore_axis_name="s"),
    compiler_params=pltpu.CompilerParams(use_tc_tiling_on_sc=False),
    out_type=jax.ShapeDtypeStruct((B * N, OUTW), jnp.float32),
    scratch_types=[
        pltpu.VMEM((C,), jnp.int32),
        pltpu.VMEM((C,), jnp.int32),
        pltpu.VMEM((C,), jnp.int32),
        pltpu.VMEM((C,), jnp.int32),
        pltpu.VMEM((C,), jnp.int32),
        pltpu.VMEM((C,), jnp.int32),
        pltpu.VMEM((C, D), jnp.float32),
        pltpu.VMEM((C, D), jnp.float32),
        pltpu.VMEM((C, D), jnp.float32),
        pltpu.VMEM((C, D), jnp.float32),
        pltpu.VMEM((C, WD), jnp.float32),
        pltpu.VMEM((C, WD), jnp.float32),
        pltpu.SemaphoreType.DMA,
        pltpu.SemaphoreType.DMA,
        pltpu.SemaphoreType.DMA,
        pltpu.SemaphoreType.DMA,
    ],
)(_body)


def kernel(sequence_tensor, span_indices, width_embedding):
    seq_flat = sequence_tensor.reshape(B * S, D)
    sp32 = span_indices.astype(jnp.int32)
    starts_flat = sp32[..., 0].reshape(B * N)
    ends_flat = sp32[..., 1].reshape(B * N)
    out = _span_kernel(seq_flat, starts_flat, ends_flat, width_embedding)
    return out.reshape(B, N, OUTW)


# speedup vs baseline: 5826.6279x; 1.5269x over previous
import functools

import jax
import jax.numpy as jnp
from jax import lax
from jax.experimental import pallas as pl
from jax.experimental.pallas import tpu as pltpu
from jax.experimental.pallas import tpu_sc as plsc

B, S, D = 16, 2048, 256
N = 4096
WD = 32
WP = 128
OUTW = 2 * D + WD

NW = 32
SPW = (B * N) // NW
C = 64
NCHUNK = SPW // C
LANES = 16


def _body(seq_hbm, starts_hbm, ends_hbm, wtab_hbm, out_hbm, wout_hbm,
          sidx, eidx, widx,
          srow0, srow1, erow0, erow1, wrow0, wrow1,
          sem_g0, sem_g1, sem_w0, sem_w1):
    cid = lax.axis_index("c")
    sid = lax.axis_index("s")
    wid = sid * 2 + cid
    rowoff = (wid // 2) * S
    first = wid * SPW

    srow = (srow0, srow1)
    erow = (erow0, erow1)
    wrow = (wrow0, wrow1)
    sem_g = (sem_g0, sem_g1)
    sem_w = (sem_w0, sem_w1)

    pltpu.sync_copy(starts_hbm.at[pl.ds(first, SPW)], sidx)
    pltpu.sync_copy(ends_hbm.at[pl.ds(first, SPW)], eidx)
    for j in range(SPW // LANES):
        sl = pl.ds(j * LANES, LANES)
        s = sidx[sl]
        e = eidx[sl]
        widx[sl] = e - s
        sidx[sl] = s + rowoff
        eidx[sl] = e + rowoff

    def fire_gathers(k, b):
        off = k * C
        pltpu.async_copy(seq_hbm.at[sidx.at[pl.ds(off, C)]], srow[b], sem_g[b])
        pltpu.async_copy(seq_hbm.at[eidx.at[pl.ds(off, C)]], erow[b], sem_g[b])
        pltpu.async_copy(wtab_hbm.at[widx.at[pl.ds(off, C)]], wrow[b],
                         sem_g[b])

    def wait_gathers(k, b):
        off = k * C
        pltpu.make_async_copy(
            seq_hbm.at[sidx.at[pl.ds(off, C)]], srow[b], sem_g[b]).wait()
        pltpu.make_async_copy(
            seq_hbm.at[eidx.at[pl.ds(off, C)]], erow[b], sem_g[b]).wait()
        pltpu.make_async_copy(
            wtab_hbm.at[widx.at[pl.ds(off, C)]], wrow[b], sem_g[b]).wait()

    def fire_writes(k, b):
        base = first + k * C
        pltpu.async_copy(srow[b], out_hbm.at[pl.ds(base, C), pl.ds(0, D)],
                         sem_w[b])
        pltpu.async_copy(erow[b], out_hbm.at[pl.ds(base, C), pl.ds(D, D)],
                         sem_w[b])
        pltpu.async_copy(wrow[b], wout_hbm.at[pl.ds(base, C)], sem_w[b])

    def wait_writes(k, b):
        base = first + k * C
        pltpu.make_async_copy(
            srow[b], out_hbm.at[pl.ds(base, C), pl.ds(0, D)], sem_w[b]).wait()
        pltpu.make_async_copy(
            erow[b], out_hbm.at[pl.ds(base, C), pl.ds(D, D)], sem_w[b]).wait()
        pltpu.make_async_copy(
            wrow[b], wout_hbm.at[pl.ds(base, C)], sem_w[b]).wait()

    fire_gathers(0, 0)
    fire_gathers(1, 1)
    wait_gathers(0, 0)
    fire_writes(0, 0)

    def pair_body(p, carry):
        k1 = 2 * p + 1
        wait_writes(k1 - 1, 0)
        fire_gathers(k1 + 1, 0)
        wait_gathers(k1, 1)
        fire_writes(k1, 1)

        k2 = 2 * p + 2
        wait_writes(k2 - 1, 1)
        fire_gathers(k2 + 1, 1)
        wait_gathers(k2, 0)
        fire_writes(k2, 0)
        return carry

    lax.fori_loop(0, (NCHUNK - 2) // 2, pair_body, 0)

    wait_gathers(NCHUNK - 1, 1)
    fire_writes(NCHUNK - 1, 1)
    wait_writes(NCHUNK - 2, 0)
    wait_writes(NCHUNK - 1, 1)


_span_kernel = functools.partial(
    pl.kernel,
    mesh=plsc.VectorSubcoreMesh(core_axis_name="c", subcore_axis_name="s"),
    out_type=(
        jax.ShapeDtypeStruct((B * N, OUTW), jnp.float32),
        jax.ShapeDtypeStruct((B * N, WP), jnp.float32),
    ),
    scratch_types=[
        pltpu.VMEM((SPW,), jnp.int32),
        pltpu.VMEM((SPW,), jnp.int32),
        pltpu.VMEM((SPW,), jnp.int32),
        pltpu.VMEM((C, D), jnp.float32),
        pltpu.VMEM((C, D), jnp.float32),
        pltpu.VMEM((C, D), jnp.float32),
        pltpu.VMEM((C, D), jnp.float32),
        pltpu.VMEM((C, WP), jnp.float32),
        pltpu.VMEM((C, WP), jnp.float32),
        pltpu.SemaphoreType.DMA,
        pltpu.SemaphoreType.DMA,
        pltpu.SemaphoreType.DMA,
        pltpu.SemaphoreType.DMA,
    ],
)(_body)


MR = 4096


def _merge_body(main_ref, wout_ref, out_ref):
    out_ref[...] = wout_ref[...]


_merge = functools.partial(
    pl.pallas_call,
    grid=((B * N) // MR,),
    in_specs=[
        pl.BlockSpec((MR, WP), lambda i: (i, (2 * D) // WP)),
        pl.BlockSpec((MR, WP), lambda i: (i, 0)),
    ],
    out_specs=pl.BlockSpec((MR, WP), lambda i: (i, (2 * D) // WP)),
    out_shape=jax.ShapeDtypeStruct((B * N, OUTW), jnp.float32),
    input_output_aliases={0: 0},
)(_merge_body)


def kernel(sequence_tensor, span_indices, width_embedding):
    seq_flat = sequence_tensor.reshape(B * S, D)
    sp32 = span_indices.astype(jnp.int32)
    starts_flat = sp32[..., 0].reshape(B * N)
    ends_flat = sp32[..., 1].reshape(B * N)
    wtab_pad = jnp.pad(width_embedding, ((0, 0), (0, WP - WD)))
    out_main, wout = _span_kernel(seq_flat, starts_flat, ends_flat, wtab_pad)
    out = _merge(out_main, wout)
    return out.reshape(B, N, OUTW)
